# Initial kernel scaffold; baseline (speedup 1.0000x reference)
import jax
import jax.numpy as jnp
from jax.experimental import pallas as pl


def kernel(x, edge_index, batch, params):
    raise NotImplementedError("write your pallas kernel here")


# speedup vs baseline: 3.1249x; 3.1249x over previous
import functools

import jax
import jax.numpy as jnp
from jax import lax
from jax.experimental import pallas as pl
from jax.experimental.pallas import tpu as pltpu
from jax.experimental.pallas import tpu_sc as plsc

N = 10000
E = 320000
D = 128
G = 64
NC, NS, L = 2, 16, 16
NW = NC * NS
EPW = E // NW
CH = 80
NCH = EPW // CH
ROWS_PT = 632
NPAD = NS * ROWS_PT
BR = NPAD // 8

_SC_MESH = plsc.VectorSubcoreMesh(
    core_axis_name="c", subcore_axis_name="s", num_cores=NC, num_subcores=NS)


def _make_edge_kernel(with_count):
    out_type = [jax.ShapeDtypeStruct((NC, NPAD, D), jnp.float32)]
    if with_count:
        out_type.append(jax.ShapeDtypeStruct((NC, NPAD, D), jnp.float32))

    def body(hj_hbm, bg_hbm, src_hbm, dst_hbm, zero_hbm, *rest):
        if with_count:
            out_hbm, cnt_hbm = rest[0], rest[1]
            rest = rest[2:]
        else:
            out_hbm = rest[0]
            rest = rest[1:]
        sidx, didx, hjb, bgb, msgb, aggsh, sem1, sem2 = rest
        c = lax.axis_index("c")
        s = lax.axis_index("s")
        wid = c * NS + s
        base = wid * EPW
        my_rows = pl.ds(s * ROWS_PT, ROWS_PT)

        def fill_msg_ones(e, carry):
            for q in range(D // L):
                msgb[e, pl.ds(q * L, L)] = jnp.ones((L,), jnp.float32)
            return carry

        if with_count:
            pltpu.sync_copy(zero_hbm, aggsh.at[my_rows])
            lax.fori_loop(0, CH, fill_msg_ones, 0)
            plsc.subcore_barrier()

            def cnt_body(k, carry):
                pltpu.sync_copy(dst_hbm.at[pl.ds(base + k * CH, CH)], didx)
                pltpu.sync_copy(msgb, aggsh.at[didx], add=True)
                return carry
            lax.fori_loop(0, NCH, cnt_body, 0)
            plsc.subcore_barrier()
            pltpu.sync_copy(aggsh.at[my_rows], cnt_hbm.at[c, my_rows])
            plsc.subcore_barrier()

        pltpu.sync_copy(zero_hbm, aggsh.at[my_rows])
        plsc.subcore_barrier()

        def chunk_body(k, carry):
            off = base + k * CH
            pltpu.sync_copy(src_hbm.at[pl.ds(off, CH)], sidx)
            pltpu.sync_copy(dst_hbm.at[pl.ds(off, CH)], didx)
            cp1 = pltpu.async_copy(hj_hbm.at[sidx], hjb, sem1)
            cp2 = pltpu.async_copy(bg_hbm.at[didx], bgb, sem2)
            cp1.wait()
            cp2.wait()

            def edge_body(e, inner):
                for q in range(D // L):
                    hv = hjb[e, pl.ds(q * L, L)]
                    bv = bgb[e, pl.ds(q * L, L)]
                    gv = bgb[e, pl.ds(D + q * L, L)]
                    msgb[e, pl.ds(q * L, L)] = jnp.maximum(gv * hv + bv, 0.0)
                return inner
            lax.fori_loop(0, CH, edge_body, 0)

            pltpu.sync_copy(msgb, aggsh.at[didx], add=True)
            return carry

        lax.fori_loop(0, NCH, chunk_body, 0)
        plsc.subcore_barrier()
        pltpu.sync_copy(aggsh.at[my_rows], out_hbm.at[c, my_rows])

    return pl.kernel(
        body,
        mesh=_SC_MESH,
        out_type=out_type,
        scratch_types=[
            pltpu.VMEM((CH,), jnp.int32),
            pltpu.VMEM((CH,), jnp.int32),
            pltpu.VMEM((CH, D), jnp.float32),
            pltpu.VMEM((CH, 2 * D), jnp.float32),
            pltpu.VMEM((CH, D), jnp.float32),
            pltpu.VMEM_SHARED((NPAD, D), jnp.float32),
            pltpu.SemaphoreType.DMA,
            pltpu.SemaphoreType.DMA,
        ],
    )


_edge_kernel0 = _make_edge_kernel(True)
_edge_kernel = _make_edge_kernel(False)


def _split_store(t, hj_ref, bg_ref, skip_ref):
    hj_ref[...] = t[:, :D]
    bg_ref[...] = t[:, D:3 * D]
    skip_ref[...] = t[:, 3 * D:]


def _dense0_body(h_ref, w_ref, b_ref, hj_ref, bg_ref, skip_ref):
    t = jnp.dot(h_ref[...], w_ref[...],
                preferred_element_type=jnp.float32) + b_ref[...]
    _split_store(t, hj_ref, bg_ref, skip_ref)


def _combine_h(p_ref, cnt_ref, skip_ref):
    agg = p_ref[0] + p_ref[1]
    cnt = jnp.maximum(cnt_ref[0][:, :1] + cnt_ref[1][:, :1], 1.0)
    return jax.nn.relu(skip_ref[...] + agg / cnt)


def _update_body(p_ref, cnt_ref, skip_ref, g_ref, b_ref, w_ref, bc_ref,
                 hj_ref, bg_ref, skip_out_ref):
    h = _combine_h(p_ref, cnt_ref, skip_ref)
    mu = jnp.mean(h, axis=-1, keepdims=True)
    dlt = h - mu
    var = jnp.mean(dlt * dlt, axis=-1, keepdims=True)
    hn = dlt * lax.rsqrt(var + 1e-5) * g_ref[...] + b_ref[...]
    t = jnp.dot(hn, w_ref[...],
                preferred_element_type=jnp.float32) + bc_ref[...]
    _split_store(t, hj_ref, bg_ref, skip_out_ref)


def _pool_body(p_ref, cnt_ref, skip_ref, batch_ref, wp_ref, bp_ref, out_ref):
    agg = p_ref[0] + p_ref[1]
    cnt = jnp.maximum(cnt_ref[0][:, :1] + cnt_ref[1][:, :1], 1.0)
    h = jax.nn.relu(skip_ref[...] + agg / cnt)
    bv = batch_ref[...]
    gids = lax.broadcasted_iota(jnp.int32, (G, NPAD), 0)
    mask = (bv == gids).astype(jnp.float32)
    ssum = jnp.dot(mask, h, preferred_element_type=jnp.float32)
    cg = jnp.sum(mask, axis=1, keepdims=True)
    pooled = ssum / jnp.maximum(cg, 1.0)
    out_ref[...] = jnp.dot(pooled, wp_ref[...],
                           preferred_element_type=jnp.float32) + bp_ref[...]


_row_spec = lambda w: pl.BlockSpec((BR, w), lambda i: (i, 0))
_full = lambda shape: pl.BlockSpec(shape, lambda i: tuple(0 for _ in shape))

_dense0 = pl.pallas_call(
    _dense0_body,
    grid=(8,),
    in_specs=[_row_spec(D), _full((D, 4 * D)), _full((1, 4 * D))],
    out_specs=[_row_spec(D), _row_spec(2 * D), _row_spec(D)],
    out_shape=[jax.ShapeDtypeStruct((NPAD, D), jnp.float32),
               jax.ShapeDtypeStruct((NPAD, 2 * D), jnp.float32),
               jax.ShapeDtypeStruct((NPAD, D), jnp.float32)],
)

_update = pl.pallas_call(
    _update_body,
    grid=(8,),
    in_specs=[pl.BlockSpec((NC, BR, D), lambda i: (0, i, 0)),
              pl.BlockSpec((NC, BR, D), lambda i: (0, i, 0)),
              _row_spec(D), _full((1, D)), _full((1, D)),
              _full((D, 4 * D)), _full((1, 4 * D))],
    out_specs=[_row_spec(D), _row_spec(2 * D), _row_spec(D)],
    out_shape=[jax.ShapeDtypeStruct((NPAD, D), jnp.float32),
               jax.ShapeDtypeStruct((NPAD, 2 * D), jnp.float32),
               jax.ShapeDtypeStruct((NPAD, D), jnp.float32)],
)

_pool = pl.pallas_call(
    _pool_body,
    in_specs=[pl.BlockSpec((NC, NPAD, D), lambda: (0, 0, 0)),
              pl.BlockSpec((NC, NPAD, D), lambda: (0, 0, 0)),
              pl.BlockSpec((NPAD, D), lambda: (0, 0)),
              pl.BlockSpec((1, NPAD), lambda: (0, 0)),
              pl.BlockSpec((D, 1), lambda: (0, 0)),
              pl.BlockSpec((1, 1), lambda: (0, 0))],
    out_specs=pl.BlockSpec((G, 1), lambda: (0, 0)),
    out_shape=jax.ShapeDtypeStruct((G, 1), jnp.float32),
)


def kernel(x, edge_index, batch, params):
    src = edge_index[0]
    dst = edge_index[1]
    xp = jnp.zeros((NPAD, D), jnp.float32).at[:N].set(x.astype(jnp.float32))
    batchf = jnp.full((1, NPAD), G, jnp.int32)
    batchf = batchf.at[0, :N].set(batch.astype(jnp.int32))
    zero_init = jnp.zeros((ROWS_PT, D), jnp.float32)

    skip = None
    p = None
    cnt = None
    for l in range(3):
        wcat = jnp.concatenate(
            [params['W_lin_%d' % l].T, params['W_film_%d' % l].T,
             params['W_skip_%d' % l].T], axis=1)
        bcat = jnp.concatenate(
            [jnp.zeros((D,), jnp.float32), params['b_film_%d' % l],
             params['b_skip_%d' % l]])[None, :]
        if l == 0:
            hj, bg, skip = _dense0(xp, wcat, bcat)
            p, cnt = jax.tree.leaves(_edge_kernel0(hj, bg, src, dst, zero_init))
        else:
            lng = params['ln%d_g' % (l - 1)][None, :]
            lnb = params['ln%d_b' % (l - 1)][None, :]
            hj, bg, skip = _update(p, cnt, skip, lng, lnb, wcat, bcat)
            p = jax.tree.leaves(_edge_kernel(hj, bg, src, dst, zero_init))[0]

    wpt = params['W_post'].T
    bp = params['b_post'][None, :]
    return _pool(p, cnt, skip, batchf, wpt, bp)
